# baseline (speedup 1.0000x reference)
import functools
import math

import jax
import jax.numpy as jnp
from jax import lax
from jax.experimental import pallas as pl
from jax.experimental.pallas import tpu as pltpu
from jax.experimental.pallas import tpu_sc as plsc

L = 16
NTILES = 16

_LN2 = 0.6931471805599453
_SQRT2 = 1.4142135623730951
_EPS = 1e-6
_SIGMA = 0.9
_C1 = -math.log(1.0 - _SIGMA)


def _vlog(x):
    bits = plsc.bitcast(x, jnp.int32)
    e = lax.shift_right_logical(bits, 23) - 127
    m = plsc.bitcast(
        (bits & jnp.int32(0x007FFFFF)) | jnp.int32(0x3F800000), jnp.float32)
    big = m > _SQRT2
    m = jnp.where(big, 0.5 * m, m)
    ef = (e + jnp.where(big, 1, 0)).astype(jnp.float32)
    s = (m - 1.0) / (m + 1.0)
    z = s * s
    p = 1.0 + z * (1.0 / 3.0 + z * (0.2 + z * (1.0 / 7.0 + z * (1.0 / 9.0))))
    return 2.0 * s * p + ef * _LN2


def _smooth_l1(d):
    ad = jnp.abs(d)
    return jnp.where(ad < 1.0, 0.5 * ad * ad, ad - 0.5)


def _sc_rep_loss(p2, t2, predM, indsB, targM, B, P, G, N):
    tiles_per_img = NTILES // B
    preds_per_tile = P // tiles_per_img
    gts_per_tile = G // tiles_per_img
    CB = -(-N // (NTILES * L)) * L
    SKIP = (NTILES * CB - N) // L
    last_start = N - CB

    mesh = plsc.VectorSubcoreMesh(
        core_axis_name="c", subcore_axis_name="s", num_cores=1)

    @functools.partial(
        pl.kernel,
        out_type=jax.ShapeDtypeStruct((L,), jnp.float32),
        mesh=mesh,
        compiler_params=pltpu.CompilerParams(
            needs_layout_passes=False, use_tc_tiling_on_sc=False),
        scratch_types=[
            pltpu.VMEM((CB, 4), jnp.float32),
            pltpu.VMEM((CB, 4), jnp.float32),
            pltpu.VMEM((P, 4), jnp.float32),
            pltpu.VMEM((1, P), jnp.int32),
            pltpu.VMEM((G, 4), jnp.float32),
            pltpu.VMEM((G,), jnp.float32),
            pltpu.VMEM((5 * G,), jnp.float32),
            pltpu.VMEM((L,), jnp.float32),
            pltpu.VMEM_SHARED((NTILES, L), jnp.float32),
            pltpu.VMEM((NTILES, L), jnp.float32),
            pltpu.VMEM((L,), jnp.float32),
            pltpu.SemaphoreType.DMA,
            pltpu.SemaphoreType.DMA,
        ],
    )
    def run(p2_hbm, t2_hbm, pred_hbm, inds_hbm, targ_hbm, out_hbm,
            p2v, t2v, predv, indsv, targv, gareav, histv, partv, sharedp,
            allpv, outv, sem, sem2):
        wid = lax.axis_index("s")
        img = wid // tiles_per_img
        q = wid % tiles_per_img
        is_last = wid == NTILES - 1
        box0 = jnp.where(is_last, last_start, wid * CB)

        big_cps = [
            pltpu.async_copy(p2_hbm.at[pl.ds(box0, CB)], p2v, sem),
            pltpu.async_copy(t2_hbm.at[pl.ds(box0, CB)], t2v, sem),
        ]
        small_cps = [
            pltpu.async_copy(pred_hbm.at[pl.ds(img * P, P)], predv, sem2),
            pltpu.async_copy(inds_hbm.at[pl.ds(img, 1)], indsv, sem2),
            pltpu.async_copy(targ_hbm.at[pl.ds(img * G, G)], targv, sem2),
        ]
        for cp in small_cps:
            cp.wait()

        zeros = jnp.zeros((L,), jnp.float32)
        ones = jnp.ones((L,), jnp.float32)
        iota = lax.broadcasted_iota(jnp.int32, (L,), 0)

        row = [jnp.full((L,), c, jnp.int32) for c in range(4)]
        for gc in range(G // L):
            rv = gc * L + iota
            gx1 = plsc.load_gather(targv, [rv, row[0]])
            gy1 = plsc.load_gather(targv, [rv, row[1]])
            gx2 = plsc.load_gather(targv, [rv, row[2]])
            gy2 = plsc.load_gather(targv, [rv, row[3]])
            gareav[pl.ds(gc * L, L)] = (gx2 - gx1) * (gy2 - gy1)

        NCH = 4
        rep_sv = zeros
        rep_nv = zeros
        for half in range(preds_per_tile // (NCH * L)):
            pdata = []
            for kc in range(NCH):
                base = q * preds_per_tile + (half * NCH + kc) * L
                bv = base + iota
                pdata.append((plsc.load_gather(predv, [bv, row[0]]),
                              plsc.load_gather(predv, [bv, row[1]]),
                              plsc.load_gather(predv, [bv, row[2]]),
                              plsc.load_gather(predv, [bv, row[3]]),
                              indsv[0, pl.ds(base, L)]))

            def gstep(g, carry):
                bests, garbs = carry
                gidx = jnp.full((L,), g, jnp.int32)
                tx1 = plsc.load_gather(targv, [gidx, row[0]])
                ty1 = plsc.load_gather(targv, [gidx, row[1]])
                tx2 = plsc.load_gather(targv, [gidx, row[2]])
                ty2 = plsc.load_gather(targv, [gidx, row[3]])
                ga = plsc.load_gather(gareav, [gidx])
                nb, ng = [], []
                for kc in range(NCH):
                    px1, py1, px2, py2, pind = pdata[kc]
                    iw = jnp.maximum(
                        jnp.minimum(px2, tx2) - jnp.maximum(px1, tx1), 0.0)
                    ih = jnp.maximum(
                        jnp.minimum(py2, ty2) - jnp.maximum(py1, ty1), 0.0)
                    ov = jnp.where(pind == gidx, 0.0, iw * ih)
                    upd = ov > bests[kc]
                    nb.append(jnp.where(upd, ov, bests[kc]))
                    ng.append(jnp.where(upd, ga, garbs[kc]))
                return tuple(nb), tuple(ng)

            bests, garbs = lax.fori_loop(
                0, G, gstep, ((zeros,) * NCH, (ones,) * NCH))
            for kc in range(NCH):
                best = bests[kc]
                valid = best > 0.0
                iog = best / garbs[kc]
                one_m = jnp.maximum(1.0 - iog, _EPS)
                sml = jnp.where(iog > _SIGMA,
                                (iog - _SIGMA) * (1.0 / (1.0 - _SIGMA)) + _C1,
                                -_vlog(one_m))
                rep_sv = rep_sv + jnp.where(valid, sml, 0.0)
                rep_nv = rep_nv + jnp.where(valid, 1.0, 0.0)
        rep_s = jnp.sum(rep_sv)
        rep_n = jnp.sum(rep_nv)

        for r in range(5):
            for c4 in range(G // L):
                histv[pl.ds(r * G + c4 * L, L)] = zeros

        def pstep(kc, _):
            o = kc * L
            ov = o + iota
            indv = indsv[0, pl.ds(o, L)]
            plsc.addupdate_scatter(histv, [indv], ones)
            plsc.addupdate_scatter(histv, [indv + G],
                                   plsc.load_gather(predv, [ov, row[0]]))
            plsc.addupdate_scatter(histv, [indv + 2 * G],
                                   plsc.load_gather(predv, [ov, row[1]]))
            plsc.addupdate_scatter(histv, [indv + 3 * G],
                                   plsc.load_gather(predv, [ov, row[2]]))
            plsc.addupdate_scatter(histv, [indv + 4 * G],
                                   plsc.load_gather(predv, [ov, row[3]]))
            return 0

        lax.fori_loop(0, P // L, pstep, 0)
        goff0 = q * gts_per_tile
        cnt = histv[pl.ds(goff0, L)]
        s1 = histv[pl.ds(G + goff0, L)]
        s2 = histv[pl.ds(2 * G + goff0, L)]
        s3 = histv[pl.ds(3 * G + goff0, L)]
        s4 = histv[pl.ds(4 * G + goff0, L)]
        cmax = jnp.maximum(cnt, 1.0)
        gv = goff0 + iota
        sl = (_smooth_l1(plsc.load_gather(targv, [gv, row[0]]) - s1 / cmax)
              + _smooth_l1(plsc.load_gather(targv, [gv, row[1]]) - s2 / cmax)
              + _smooth_l1(plsc.load_gather(targv, [gv, row[2]]) - s3 / cmax)
              + _smooth_l1(plsc.load_gather(targv, [gv, row[3]]) - s4 / cmax)
              ) * 0.25
        gm = cnt > 1.0
        com_s = jnp.sum(jnp.where(gm, sl, 0.0))
        com_n = jnp.sum(jnp.where(gm, 1.0, 0.0))

        for cp in big_cps:
            cp.wait()

        def iou_step(k, acc):
            rv = k * L + iota
            px1 = plsc.load_gather(p2v, [rv, row[0]])
            py1 = plsc.load_gather(p2v, [rv, row[1]])
            px2 = plsc.load_gather(p2v, [rv, row[2]])
            py2 = plsc.load_gather(p2v, [rv, row[3]])
            tx1 = plsc.load_gather(t2v, [rv, row[0]])
            ty1 = plsc.load_gather(t2v, [rv, row[1]])
            tx2 = plsc.load_gather(t2v, [rv, row[2]])
            ty2 = plsc.load_gather(t2v, [rv, row[3]])
            w = jnp.maximum(jnp.minimum(px2, tx2) - jnp.maximum(px1, tx1), 0.0)
            h = jnp.maximum(jnp.minimum(py2, ty2) - jnp.maximum(py1, ty1), 0.0)
            ov = w * h
            ap = (px2 - px1) * (py2 - py1)
            ag = (tx2 - tx1) * (ty2 - ty1)
            union = jnp.maximum(ap + ag - ov, _EPS)
            iou = jnp.maximum(ov / union, _EPS)
            ok = jnp.logical_or(jnp.logical_not(is_last), k >= SKIP)
            return acc + jnp.where(ok, -_vlog(iou), zeros)

        iou_acc = lax.fori_loop(0, CB // L, iou_step, zeros, unroll=2)
        iou_s = jnp.sum(iou_acc)

        iv = iota
        pvec = (jnp.where(iv == 0, iou_s, 0.0)
                + jnp.where(iv == 1, rep_s, 0.0)
                + jnp.where(iv == 2, rep_n, 0.0)
                + jnp.where(iv == 3, com_s, 0.0)
                + jnp.where(iv == 4, com_n, 0.0))
        partv[...] = pvec
        pltpu.sync_copy(partv, sharedp.at[wid])
        plsc.subcore_barrier()

        @pl.when(wid == 0)
        def _finalize():
            pltpu.sync_copy(sharedp, allpv)
            acc = zeros
            for i in range(NTILES):
                acc = acc + allpv[i]
            t_iou = jnp.broadcast_to(acc[0], (L,))
            t_rep_s = jnp.broadcast_to(acc[1], (L,))
            t_rep_n = jnp.broadcast_to(acc[2], (L,))
            t_com_s = jnp.broadcast_to(acc[3], (L,))
            t_com_n = jnp.broadcast_to(acc[4], (L,))
            rep = jnp.where(t_rep_n > 0.0,
                            10.0 * t_rep_s / jnp.maximum(t_rep_n, 1.0), 0.0)
            com = jnp.where(t_com_n > 0.0,
                            10.0 * t_com_s / jnp.maximum(t_com_n, 1.0), 0.0)
            total = t_iou * (1.0 / N) + rep + com
            outv[...] = jnp.where(iv == 0, total, 0.0)
            pltpu.sync_copy(outv, out_hbm)

    return run(p2, t2, predM, indsB, targM)


def kernel(pred, pos_assigned_gt_inds, target, pred2, target2):
    B, P, _ = pred.shape
    G = target.shape[1]
    N = pred2.shape[0]
    out = _sc_rep_loss(
        pred2, target2,
        pred.reshape(B * P, 4),
        pos_assigned_gt_inds.astype(jnp.int32),
        target.reshape(B * G, 4),
        B, P, G, N)
    return out[0]


# speedup vs baseline: 2.4408x; 2.4408x over previous
import functools
import math

import jax
import jax.numpy as jnp
from jax import lax
from jax.experimental import pallas as pl
from jax.experimental.pallas import tpu as pltpu
from jax.experimental.pallas import tpu_sc as plsc

L = 16
NTILES = 16

_LN2 = 0.6931471805599453
_SQRT2 = 1.4142135623730951
_EPS = 1e-6
_SIGMA = 0.9
_C1 = -math.log(1.0 - _SIGMA)


def _vlog(x):
    bits = plsc.bitcast(x, jnp.int32)
    e = lax.shift_right_logical(bits, 23) - 127
    m = plsc.bitcast(
        (bits & jnp.int32(0x007FFFFF)) | jnp.int32(0x3F800000), jnp.float32)
    big = m > _SQRT2
    m = jnp.where(big, 0.5 * m, m)
    ef = (e + jnp.where(big, 1, 0)).astype(jnp.float32)
    s = (m - 1.0) / (m + 1.0)
    z = s * s
    p = 1.0 + z * (1.0 / 3.0 + z * (0.2 + z * (1.0 / 7.0 + z * (1.0 / 9.0))))
    return 2.0 * s * p + ef * _LN2


def _smooth_l1(d):
    ad = jnp.abs(d)
    return jnp.where(ad < 1.0, 0.5 * ad * ad, ad - 0.5)


def _sc_rep_loss(cols_flat, predT2, indsF, targT2, B, P, G, N, NPAD):
    CHUNK = NPAD // NTILES
    tiles_per_img = NTILES // B
    preds_per_tile = P // tiles_per_img
    gts_per_tile = G // tiles_per_img

    mesh = plsc.VectorSubcoreMesh(
        core_axis_name="c", subcore_axis_name="s", num_cores=2)

    @functools.partial(
        pl.kernel,
        out_type=jax.ShapeDtypeStruct((2, L), jnp.float32),
        mesh=mesh,
        compiler_params=pltpu.CompilerParams(
            needs_layout_passes=False, use_tc_tiling_on_sc=False),
        scratch_types=[
            pltpu.VMEM((8, CHUNK), jnp.float32),
            pltpu.VMEM((4, P), jnp.float32),
            pltpu.VMEM((P,), jnp.int32),
            pltpu.VMEM((4, G), jnp.float32),
            pltpu.VMEM((G,), jnp.float32),
            pltpu.VMEM((5 * G,), jnp.float32),
            pltpu.VMEM((L,), jnp.float32),
            pltpu.VMEM_SHARED((NTILES, L), jnp.float32),
            pltpu.VMEM((NTILES, L), jnp.float32),
            pltpu.VMEM((L,), jnp.float32),
            pltpu.SemaphoreType.DMA,
            pltpu.SemaphoreType.DMA,
        ],
    )
    def run(cols_hbm, pred_hbm, inds_hbm, targ_hbm, out_hbm,
            colsv, predv, indsv, targv, gareav, histv, partv, sharedp, allpv,
            outv, sem, sem2):
        cid = lax.axis_index("c")
        wid = lax.axis_index("s")

        zeros = jnp.zeros((L,), jnp.float32)
        ones = jnp.ones((L,), jnp.float32)
        iota = lax.broadcasted_iota(jnp.int32, (L,), 0)
        row = [jnp.full((L,), c, jnp.int32) for c in range(4)]

        @pl.when(cid == 1)
        def _iou_core():
            cps = []
            for c in range(8):
                cps.append(pltpu.async_copy(
                    cols_hbm.at[pl.ds(c * NPAD + wid * CHUNK, CHUNK)],
                    colsv.at[c], sem))
            for cp in cps:
                cp.wait()

            def iou_step(k, acc):
                o = k * L
                px1 = colsv[0, pl.ds(o, L)]
                py1 = colsv[1, pl.ds(o, L)]
                px2 = colsv[2, pl.ds(o, L)]
                py2 = colsv[3, pl.ds(o, L)]
                tx1 = colsv[4, pl.ds(o, L)]
                ty1 = colsv[5, pl.ds(o, L)]
                tx2 = colsv[6, pl.ds(o, L)]
                ty2 = colsv[7, pl.ds(o, L)]
                w = jnp.maximum(
                    jnp.minimum(px2, tx2) - jnp.maximum(px1, tx1), 0.0)
                h = jnp.maximum(
                    jnp.minimum(py2, ty2) - jnp.maximum(py1, ty1), 0.0)
                ov = w * h
                ap = (px2 - px1) * (py2 - py1)
                ag = (tx2 - tx1) * (ty2 - ty1)
                union = jnp.maximum(ap + ag - ov, _EPS)
                iou = jnp.maximum(ov / union, _EPS)
                return acc - _vlog(iou)

            iou_acc = lax.fori_loop(0, CHUNK // L, iou_step, zeros, unroll=2)
            partv[...] = iou_acc * (1.0 / N)
            pltpu.sync_copy(partv, sharedp.at[wid])
            plsc.subcore_barrier()

            @pl.when(wid == 0)
            def _fin1():
                pltpu.sync_copy(sharedp, allpv)
                acc = zeros
                for i in range(NTILES):
                    acc = acc + allpv[i]
                tot = jnp.broadcast_to(jnp.sum(acc), (L,))
                outv[...] = jnp.where(iota == 0, tot, 0.0)
                pltpu.sync_copy(outv, out_hbm.at[1])

        @pl.when(cid == 0)
        def _rep_core():
            img = wid // tiles_per_img
            q = wid % tiles_per_img
            cps = [
                pltpu.async_copy(
                    pred_hbm.at[pl.ds(img * 4, 4)], predv, sem2),
                pltpu.async_copy(
                    inds_hbm.at[pl.ds(img * P, P)], indsv, sem2),
                pltpu.async_copy(
                    targ_hbm.at[pl.ds(img * 4, 4)], targv, sem2),
            ]
            for cp in cps:
                cp.wait()

            for gc in range(G // L):
                gareav[pl.ds(gc * L, L)] = (
                    (targv[2, pl.ds(gc * L, L)] - targv[0, pl.ds(gc * L, L)])
                    * (targv[3, pl.ds(gc * L, L)]
                       - targv[1, pl.ds(gc * L, L)]))

            NCH = 4
            rep_sv = zeros
            rep_nv = zeros
            for half in range(preds_per_tile // (NCH * L)):
                pdata = []
                for kc in range(NCH):
                    base = q * preds_per_tile + (half * NCH + kc) * L
                    pdata.append((predv[0, pl.ds(base, L)],
                                  predv[1, pl.ds(base, L)],
                                  predv[2, pl.ds(base, L)],
                                  predv[3, pl.ds(base, L)],
                                  indsv[pl.ds(base, L)]))

                def gstep(g, carry):
                    bests, garbs = carry
                    gidx = jnp.full((L,), g, jnp.int32)
                    tx1 = plsc.load_gather(targv, [row[0], gidx])
                    ty1 = plsc.load_gather(targv, [row[1], gidx])
                    tx2 = plsc.load_gather(targv, [row[2], gidx])
                    ty2 = plsc.load_gather(targv, [row[3], gidx])
                    ga = plsc.load_gather(gareav, [gidx])
                    nb, ng = [], []
                    for kc in range(NCH):
                        px1, py1, px2, py2, pind = pdata[kc]
                        iw = jnp.maximum(
                            jnp.minimum(px2, tx2) - jnp.maximum(px1, tx1),
                            0.0)
                        ih = jnp.maximum(
                            jnp.minimum(py2, ty2) - jnp.maximum(py1, ty1),
                            0.0)
                        ov = jnp.where(pind == gidx, 0.0, iw * ih)
                        upd = ov > bests[kc]
                        nb.append(jnp.where(upd, ov, bests[kc]))
                        ng.append(jnp.where(upd, ga, garbs[kc]))
                    return tuple(nb), tuple(ng)

                bests, garbs = lax.fori_loop(
                    0, G, gstep, ((zeros,) * NCH, (ones,) * NCH))
                for kc in range(NCH):
                    best = bests[kc]
                    valid = best > 0.0
                    iog = best / garbs[kc]
                    one_m = jnp.maximum(1.0 - iog, _EPS)
                    sml = jnp.where(
                        iog > _SIGMA,
                        (iog - _SIGMA) * (1.0 / (1.0 - _SIGMA)) + _C1,
                        -_vlog(one_m))
                    rep_sv = rep_sv + jnp.where(valid, sml, 0.0)
                    rep_nv = rep_nv + jnp.where(valid, 1.0, 0.0)
            rep_s = jnp.sum(rep_sv)
            rep_n = jnp.sum(rep_nv)

            for r in range(5):
                for c4 in range(G // L):
                    histv[pl.ds(r * G + c4 * L, L)] = zeros

            def pstep(kc, _):
                o = kc * L
                indv = indsv[pl.ds(o, L)]
                plsc.addupdate_scatter(histv, [indv], ones)
                plsc.addupdate_scatter(histv, [indv + G],
                                       predv[0, pl.ds(o, L)])
                plsc.addupdate_scatter(histv, [indv + 2 * G],
                                       predv[1, pl.ds(o, L)])
                plsc.addupdate_scatter(histv, [indv + 3 * G],
                                       predv[2, pl.ds(o, L)])
                plsc.addupdate_scatter(histv, [indv + 4 * G],
                                       predv[3, pl.ds(o, L)])
                return 0

            lax.fori_loop(0, P // L, pstep, 0)
            goff0 = q * gts_per_tile
            cnt = histv[pl.ds(goff0, L)]
            s1 = histv[pl.ds(G + goff0, L)]
            s2 = histv[pl.ds(2 * G + goff0, L)]
            s3 = histv[pl.ds(3 * G + goff0, L)]
            s4 = histv[pl.ds(4 * G + goff0, L)]
            cmax = jnp.maximum(cnt, 1.0)
            sl = (_smooth_l1(targv[0, pl.ds(goff0, L)] - s1 / cmax)
                  + _smooth_l1(targv[1, pl.ds(goff0, L)] - s2 / cmax)
                  + _smooth_l1(targv[2, pl.ds(goff0, L)] - s3 / cmax)
                  + _smooth_l1(targv[3, pl.ds(goff0, L)] - s4 / cmax)) * 0.25
            gm = cnt > 1.0
            com_s = jnp.sum(jnp.where(gm, sl, 0.0))
            com_n = jnp.sum(jnp.where(gm, 1.0, 0.0))

            pvec = (jnp.where(iota == 0, rep_s, 0.0)
                    + jnp.where(iota == 1, rep_n, 0.0)
                    + jnp.where(iota == 2, com_s, 0.0)
                    + jnp.where(iota == 3, com_n, 0.0))
            partv[...] = pvec
            pltpu.sync_copy(partv, sharedp.at[wid])
            plsc.subcore_barrier()

            @pl.when(wid == 0)
            def _fin0():
                pltpu.sync_copy(sharedp, allpv)
                acc = zeros
                for i in range(NTILES):
                    acc = acc + allpv[i]
                t_rep_s = jnp.broadcast_to(acc[0], (L,))
                t_rep_n = jnp.broadcast_to(acc[1], (L,))
                t_com_s = jnp.broadcast_to(acc[2], (L,))
                t_com_n = jnp.broadcast_to(acc[3], (L,))
                rep = jnp.where(
                    t_rep_n > 0.0,
                    10.0 * t_rep_s / jnp.maximum(t_rep_n, 1.0), 0.0)
                com = jnp.where(
                    t_com_n > 0.0,
                    10.0 * t_com_s / jnp.maximum(t_com_n, 1.0), 0.0)
                outv[...] = jnp.where(iota == 0, rep + com, 0.0)
                pltpu.sync_copy(outv, out_hbm.at[0])

    return run(cols_flat, predT2, indsF, targT2)


def kernel(pred, pos_assigned_gt_inds, target, pred2, target2):
    B, P, _ = pred.shape
    G = target.shape[1]
    N = pred2.shape[0]
    NPAD = -(-N // (NTILES * L)) * (NTILES * L)

    cols = jnp.concatenate([pred2.T, target2.T], axis=0)
    if NPAD > N:
        padcol = jnp.array([0, 0, 1, 1, 0, 0, 1, 1], jnp.float32)[:, None]
        cols = jnp.concatenate(
            [cols, jnp.broadcast_to(padcol, (8, NPAD - N))], axis=1)
    cols_flat = cols.reshape(8 * NPAD)

    predT2 = pred.transpose(0, 2, 1).reshape(B * 4, P)
    targT2 = target.transpose(0, 2, 1).reshape(B * 4, G)
    indsF = pos_assigned_gt_inds.astype(jnp.int32).reshape(B * P)

    out = _sc_rep_loss(cols_flat, predT2, indsF, targT2, B, P, G, N, NPAD)
    return out[0, 0] + out[1, 0]


# speedup vs baseline: 2.7693x; 1.1346x over previous
import functools
import math

import jax
import jax.numpy as jnp
from jax import lax
from jax.experimental import pallas as pl
from jax.experimental.pallas import tpu as pltpu
from jax.experimental.pallas import tpu_sc as plsc

L = 16
NTILES = 16

_LN2 = 0.6931471805599453
_SQRT2 = 1.4142135623730951
_EPS = 1e-6
_SIGMA = 0.9
_C1 = -math.log(1.0 - _SIGMA)


def _vlog(x):
    bits = plsc.bitcast(x, jnp.int32)
    e = lax.shift_right_logical(bits, 23) - 127
    m = plsc.bitcast(
        (bits & jnp.int32(0x007FFFFF)) | jnp.int32(0x3F800000), jnp.float32)
    big = m > _SQRT2
    m = jnp.where(big, 0.5 * m, m)
    ef = (e + jnp.where(big, 1, 0)).astype(jnp.float32)
    s = (m - 1.0) / (m + 1.0)
    z = s * s
    p = 1.0 + z * (1.0 / 3.0 + z * (0.2 + z * (1.0 / 7.0 + z * (1.0 / 9.0))))
    return 2.0 * s * p + ef * _LN2


def _smooth_l1(d):
    ad = jnp.abs(d)
    return jnp.where(ad < 1.0, 0.5 * ad * ad, ad - 0.5)


def _sc_rep_loss(cols_flat, predT2, indsF, targT2, B, P, G, N, NPAD):
    CHUNK = NPAD // NTILES
    tiles_per_img = NTILES // B
    preds_per_tile = P // tiles_per_img
    gts_per_tile = G // tiles_per_img

    mesh = plsc.VectorSubcoreMesh(
        core_axis_name="c", subcore_axis_name="s", num_cores=1)

    @functools.partial(
        pl.kernel,
        out_type=jax.ShapeDtypeStruct((L,), jnp.float32),
        mesh=mesh,
        compiler_params=pltpu.CompilerParams(
            needs_layout_passes=False, use_tc_tiling_on_sc=False),
        scratch_types=[
            pltpu.VMEM((8, CHUNK), jnp.float32),
            pltpu.VMEM((4, P), jnp.float32),
            pltpu.VMEM((P,), jnp.int32),
            pltpu.VMEM((4, G), jnp.float32),
            pltpu.VMEM((G,), jnp.float32),
            pltpu.VMEM((5 * G,), jnp.float32),
            pltpu.VMEM((L,), jnp.float32),
            pltpu.VMEM_SHARED((NTILES, L), jnp.float32),
            pltpu.VMEM((NTILES, L), jnp.float32),
            pltpu.VMEM((L,), jnp.float32),
            pltpu.SemaphoreType.DMA,
            pltpu.SemaphoreType.DMA,
        ],
    )
    def run(cols_hbm, pred_hbm, inds_hbm, targ_hbm, out_hbm,
            colsv, predv, indsv, targv, gareav, histv, partv, sharedp, allpv,
            outv, sem, sem2):
        wid = lax.axis_index("s")
        img = wid // tiles_per_img
        q = wid % tiles_per_img

        cols_cps = []
        for c in range(8):
            cols_cps.append(pltpu.async_copy(
                cols_hbm.at[pl.ds(c * NPAD + wid * CHUNK, CHUNK)],
                colsv.at[c], sem))
        small_cps = [
            pltpu.async_copy(pred_hbm.at[pl.ds(img * 4, 4)], predv, sem2),
            pltpu.async_copy(inds_hbm.at[pl.ds(img * P, P)], indsv, sem2),
            pltpu.async_copy(targ_hbm.at[pl.ds(img * 4, 4)], targv, sem2),
        ]
        for cp in small_cps:
            cp.wait()

        zeros = jnp.zeros((L,), jnp.float32)
        ones = jnp.ones((L,), jnp.float32)
        iota = lax.broadcasted_iota(jnp.int32, (L,), 0)

        for gc in range(G // L):
            gareav[pl.ds(gc * L, L)] = (
                (targv[2, pl.ds(gc * L, L)] - targv[0, pl.ds(gc * L, L)])
                * (targv[3, pl.ds(gc * L, L)] - targv[1, pl.ds(gc * L, L)]))
        row = [jnp.full((L,), c, jnp.int32) for c in range(4)]

        NCH = 4
        rep_sv = zeros
        rep_nv = zeros
        for half in range(preds_per_tile // (NCH * L)):
            pdata = []
            for kc in range(NCH):
                base = q * preds_per_tile + (half * NCH + kc) * L
                pdata.append((predv[0, pl.ds(base, L)],
                              predv[1, pl.ds(base, L)],
                              predv[2, pl.ds(base, L)],
                              predv[3, pl.ds(base, L)],
                              indsv[pl.ds(base, L)]))

            def gstep(g, carry):
                bests, garbs = carry
                gidx = jnp.full((L,), g, jnp.int32)
                tx1 = plsc.load_gather(targv, [row[0], gidx])
                ty1 = plsc.load_gather(targv, [row[1], gidx])
                tx2 = plsc.load_gather(targv, [row[2], gidx])
                ty2 = plsc.load_gather(targv, [row[3], gidx])
                ga = plsc.load_gather(gareav, [gidx])
                nb, ng = [], []
                for kc in range(NCH):
                    px1, py1, px2, py2, pind = pdata[kc]
                    iw = jnp.maximum(
                        jnp.minimum(px2, tx2) - jnp.maximum(px1, tx1), 0.0)
                    ih = jnp.maximum(
                        jnp.minimum(py2, ty2) - jnp.maximum(py1, ty1), 0.0)
                    ov = jnp.where(pind == gidx, 0.0, iw * ih)
                    upd = ov > bests[kc]
                    nb.append(jnp.where(upd, ov, bests[kc]))
                    ng.append(jnp.where(upd, ga, garbs[kc]))
                return tuple(nb), tuple(ng)

            bests, garbs = lax.fori_loop(
                0, G, gstep, ((zeros,) * NCH, (ones,) * NCH))
            for kc in range(NCH):
                best = bests[kc]
                valid = best > 0.0
                iog = best / garbs[kc]
                one_m = jnp.maximum(1.0 - iog, _EPS)
                sml = jnp.where(iog > _SIGMA,
                                (iog - _SIGMA) * (1.0 / (1.0 - _SIGMA)) + _C1,
                                -_vlog(one_m))
                rep_sv = rep_sv + jnp.where(valid, sml, 0.0)
                rep_nv = rep_nv + jnp.where(valid, 1.0, 0.0)
        rep_s = jnp.sum(rep_sv)
        rep_n = jnp.sum(rep_nv)

        for r in range(5):
            for c4 in range(G // L):
                histv[pl.ds(r * G + c4 * L, L)] = zeros

        def pstep(kc, _):
            o = kc * L
            indv = indsv[pl.ds(o, L)]
            plsc.addupdate_scatter(histv, [indv], ones)
            plsc.addupdate_scatter(histv, [indv + G], predv[0, pl.ds(o, L)])
            plsc.addupdate_scatter(histv, [indv + 2 * G],
                                   predv[1, pl.ds(o, L)])
            plsc.addupdate_scatter(histv, [indv + 3 * G],
                                   predv[2, pl.ds(o, L)])
            plsc.addupdate_scatter(histv, [indv + 4 * G],
                                   predv[3, pl.ds(o, L)])
            return 0

        lax.fori_loop(0, P // L, pstep, 0)
        goff0 = q * gts_per_tile
        cnt = histv[pl.ds(goff0, L)]
        s1 = histv[pl.ds(G + goff0, L)]
        s2 = histv[pl.ds(2 * G + goff0, L)]
        s3 = histv[pl.ds(3 * G + goff0, L)]
        s4 = histv[pl.ds(4 * G + goff0, L)]
        cmax = jnp.maximum(cnt, 1.0)
        sl = (_smooth_l1(targv[0, pl.ds(goff0, L)] - s1 / cmax)
              + _smooth_l1(targv[1, pl.ds(goff0, L)] - s2 / cmax)
              + _smooth_l1(targv[2, pl.ds(goff0, L)] - s3 / cmax)
              + _smooth_l1(targv[3, pl.ds(goff0, L)] - s4 / cmax)) * 0.25
        gm = cnt > 1.0
        com_s = jnp.sum(jnp.where(gm, sl, 0.0))
        com_n = jnp.sum(jnp.where(gm, 1.0, 0.0))

        for cp in cols_cps:
            cp.wait()

        def iou_step(k, acc):
            o = k * L
            px1 = colsv[0, pl.ds(o, L)]
            py1 = colsv[1, pl.ds(o, L)]
            px2 = colsv[2, pl.ds(o, L)]
            py2 = colsv[3, pl.ds(o, L)]
            tx1 = colsv[4, pl.ds(o, L)]
            ty1 = colsv[5, pl.ds(o, L)]
            tx2 = colsv[6, pl.ds(o, L)]
            ty2 = colsv[7, pl.ds(o, L)]
            w = jnp.maximum(jnp.minimum(px2, tx2) - jnp.maximum(px1, tx1), 0.0)
            h = jnp.maximum(jnp.minimum(py2, ty2) - jnp.maximum(py1, ty1), 0.0)
            ov = w * h
            ap = (px2 - px1) * (py2 - py1)
            ag = (tx2 - tx1) * (ty2 - ty1)
            union = jnp.maximum(ap + ag - ov, _EPS)
            iou = jnp.maximum(ov / union, _EPS)
            return acc - _vlog(iou)

        iou_acc = lax.fori_loop(0, CHUNK // L, iou_step, zeros, unroll=2)
        iou_s = jnp.sum(iou_acc)

        iv = iota
        pvec = (jnp.where(iv == 0, iou_s, 0.0)
                + jnp.where(iv == 1, rep_s, 0.0)
                + jnp.where(iv == 2, rep_n, 0.0)
                + jnp.where(iv == 3, com_s, 0.0)
                + jnp.where(iv == 4, com_n, 0.0))
        partv[...] = pvec
        pltpu.sync_copy(partv, sharedp.at[wid])
        plsc.subcore_barrier()

        @pl.when(wid == 0)
        def _finalize():
            pltpu.sync_copy(sharedp, allpv)
            acc = zeros
            for i in range(NTILES):
                acc = acc + allpv[i]
            t_iou = jnp.broadcast_to(acc[0], (L,))
            t_rep_s = jnp.broadcast_to(acc[1], (L,))
            t_rep_n = jnp.broadcast_to(acc[2], (L,))
            t_com_s = jnp.broadcast_to(acc[3], (L,))
            t_com_n = jnp.broadcast_to(acc[4], (L,))
            rep = jnp.where(t_rep_n > 0.0,
                            10.0 * t_rep_s / jnp.maximum(t_rep_n, 1.0), 0.0)
            com = jnp.where(t_com_n > 0.0,
                            10.0 * t_com_s / jnp.maximum(t_com_n, 1.0), 0.0)
            total = t_iou * (1.0 / N) + rep + com
            outv[...] = jnp.where(iv == 0, total, 0.0)
            pltpu.sync_copy(outv, out_hbm)

    return run(cols_flat, predT2, indsF, targT2)


def kernel(pred, pos_assigned_gt_inds, target, pred2, target2):
    B, P, _ = pred.shape
    G = target.shape[1]
    N = pred2.shape[0]
    NPAD = -(-N // (NTILES * L)) * (NTILES * L)

    cols = jnp.concatenate([pred2.T, target2.T], axis=0)
    if NPAD > N:
        padcol = jnp.array([0, 0, 1, 1, 0, 0, 1, 1], jnp.float32)[:, None]
        cols = jnp.concatenate(
            [cols, jnp.broadcast_to(padcol, (8, NPAD - N))], axis=1)
    cols_flat = cols.reshape(8 * NPAD)

    predT2 = pred.transpose(0, 2, 1).reshape(B * 4, P)
    targT2 = target.transpose(0, 2, 1).reshape(B * 4, G)
    indsF = pos_assigned_gt_inds.astype(jnp.int32).reshape(B * P)

    out = _sc_rep_loss(cols_flat, predT2, indsF, targT2, B, P, G, N, NPAD)
    return out[0]
